# Initial kernel scaffold; baseline (speedup 1.0000x reference)
import jax
import jax.numpy as jnp
from jax.experimental import pallas as pl


def kernel(x, batch, ptr, W, b):
    raise NotImplementedError("write your pallas kernel here")


# speedup vs baseline: 7.6559x; 7.6559x over previous
import functools

import jax
import jax.numpy as jnp
from jax.experimental import pallas as pl
from jax.experimental.pallas import tpu as pltpu

N = 32768
B = 16
D = 768
BLOCK_R = 2048


def _fused_body(x_ref, batch_ref, w_ref, b_ref, pooled_ref, weights_ref, ratio_ref):
    i = pl.program_id(0)
    nsteps = pl.num_programs(0)

    xb = x_ref[...]
    z = jax.lax.dot_general(
        xb, w_ref[...], (((1,), (0,)), ((), ())),
        preferred_element_type=jnp.float32,
    ) + b_ref[0, 0]
    w = jax.nn.sigmoid(z)
    weights_ref[...] = w
    xw = xb * w

    seg = batch_ref[...]
    onehot = (seg == jax.lax.broadcasted_iota(jnp.int32, (BLOCK_R, B), 1)
              ).astype(jnp.float32)
    partial = jax.lax.dot_general(
        onehot, xw, (((0,), (0,)), ((), ())),
        preferred_element_type=jnp.float32,
    )
    cnt = jnp.sum((z >= 0.0).astype(jnp.float32)).reshape(1, 1)

    @pl.when(i == 0)
    def _init():
        pooled_ref[...] = jnp.zeros_like(pooled_ref)
        ratio_ref[...] = jnp.zeros((1, 1), jnp.float32)

    pooled_ref[...] += partial
    ratio_ref[...] += cnt

    @pl.when(i == nsteps - 1)
    def _fin():
        ratio_ref[...] = ratio_ref[...] * (1.0 / N)


def kernel(x, batch, ptr, W, b):
    del ptr
    batch2 = batch.reshape(N, 1)
    b2 = b.reshape(1, 1)
    grid = (N // BLOCK_R,)
    pooled, weights, ratio = pl.pallas_call(
        _fused_body,
        grid=grid,
        in_specs=[
            pl.BlockSpec((BLOCK_R, D), lambda i: (i, 0)),
            pl.BlockSpec((BLOCK_R, 1), lambda i: (i, 0)),
            pl.BlockSpec((D, 1), lambda i: (0, 0)),
            pl.BlockSpec((1, 1), lambda i: (0, 0)),
        ],
        out_specs=[
            pl.BlockSpec((B, D), lambda i: (0, 0)),
            pl.BlockSpec((BLOCK_R, 1), lambda i: (i, 0)),
            pl.BlockSpec((1, 1), lambda i: (0, 0)),
        ],
        out_shape=[
            jax.ShapeDtypeStruct((B, D), jnp.float32),
            jax.ShapeDtypeStruct((N, 1), jnp.float32),
            jax.ShapeDtypeStruct((1, 1), jnp.float32),
        ],
        compiler_params=pltpu.CompilerParams(
            dimension_semantics=("arbitrary",),
        ),
    )(x, batch2, W, b2)
    return pooled, weights, ratio.reshape(())
